# baseline (speedup 1.0000x reference)
import jax
import jax.numpy as jnp
from jax.experimental import pallas as pl
from jax.experimental.pallas import tpu as pltpu

_CH = 1792
_NBUF = 5


def _chunk_schedule(M):
    head = [256, 256, 512, 1024]
    tail = [1024, 512, 256, 256]
    mid_total = M - sum(head) - sum(tail)
    if mid_total > 0 and mid_total % _CH == 0:
        sizes = head + [_CH] * (mid_total // _CH) + tail
    elif M % _CH == 0:
        sizes = [_CH] * (M // _CH)
    else:
        sizes = [M]
    offs, o = [], 0
    for s in sizes:
        offs.append(o)
        o += s
    return list(zip(offs, sizes))


def _make_body(schedule):
    n = len(schedule)

    def body(x_hbm, w_hbm, b_hbm, o_hbm,
             xbuf, obuf, wv, bv, load_sem, store_sem, w_sem, b_sem):
        def load(i, slot):
            off, sz = schedule[i]
            return pltpu.make_async_copy(
                x_hbm.at[pl.ds(off, sz), :], xbuf.at[slot, pl.ds(0, sz), :],
                load_sem.at[slot])

        def store(i, slot):
            off, sz = schedule[i]
            return pltpu.make_async_copy(
                obuf.at[slot, pl.ds(0, sz), :], o_hbm.at[pl.ds(off, sz), :],
                store_sem.at[slot])

        pltpu.make_async_copy(w_hbm, wv, w_sem).start()
        pltpu.make_async_copy(b_hbm, bv, b_sem).start()
        for i in range(min(_NBUF, n)):
            load(i, i % _NBUF).start()
        pltpu.make_async_copy(w_hbm, wv, w_sem).wait()
        pltpu.make_async_copy(b_hbm, bv, b_sem).wait()

        for i in range(n):
            slot = i % _NBUF
            sz = schedule[i][1]
            load(i, slot).wait()
            if i >= _NBUF:
                store(i - _NBUF, slot).wait()
            acc = jax.lax.dot_general(
                xbuf[slot, 0:sz, :], wv[...],
                dimension_numbers=(((1,), (1,)), ((), ())),
                preferred_element_type=jnp.float32,
            )
            obuf[slot, 0:sz, :] = acc + bv[...]
            store(i, slot).start()
            nxt = i + _NBUF
            if nxt < n:
                load(nxt, slot).start()

        for i in range(max(0, n - _NBUF), n):
            store(i, i % _NBUF).wait()

    return body


def kernel(src, W, b):
    B, S, D = src.shape
    E = W.shape[0]
    M = B * S
    x = src.reshape(M, D)
    schedule = _chunk_schedule(M)
    bufrows = max(sz for _, sz in schedule)

    out = pl.pallas_call(
        _make_body(schedule),
        in_specs=[
            pl.BlockSpec(memory_space=pltpu.HBM),
            pl.BlockSpec(memory_space=pltpu.HBM),
            pl.BlockSpec(memory_space=pltpu.HBM),
        ],
        out_specs=pl.BlockSpec(memory_space=pltpu.HBM),
        out_shape=jax.ShapeDtypeStruct((M, E), jnp.float32),
        scratch_shapes=[
            pltpu.VMEM((_NBUF, bufrows, D), jnp.float32),
            pltpu.VMEM((_NBUF, bufrows, E), jnp.float32),
            pltpu.VMEM((E, D), jnp.float32),
            pltpu.VMEM((1, E), jnp.float32),
            pltpu.SemaphoreType.DMA((_NBUF,)),
            pltpu.SemaphoreType.DMA((_NBUF,)),
            pltpu.SemaphoreType.DMA,
            pltpu.SemaphoreType.DMA,
        ],
    )(x, W, b.reshape(1, E))
    return out.reshape(B, S, E)


# speedup vs baseline: 1.0174x; 1.0174x over previous
import jax
import jax.numpy as jnp
from jax.experimental import pallas as pl
from jax.experimental.pallas import tpu as pltpu

_CH = 2048
_NBUF = 4


def _chunk_schedule(M):
    head = [256, 256, 512, 1024]
    tail = [1024, 512, 256, 256]
    mid_total = M - sum(head) - sum(tail)
    if mid_total > 0 and mid_total % _CH == 0:
        sizes = head + [_CH] * (mid_total // _CH) + tail
    elif M % _CH == 0:
        sizes = [_CH] * (M // _CH)
    else:
        sizes = [M]
    offs, o = [], 0
    for s in sizes:
        offs.append(o)
        o += s
    return list(zip(offs, sizes))


def _make_body(schedule, E):
    n = len(schedule)
    Eh = E // 2
    dims = (((1,), (1,)), ((), ()))

    def body(x_hbm, w_hbm, b_hbm, o_hbm,
             xbuf, obuf, wv, bv, load_sem, store_sem, w_sem, b_sem):
        def load(i, slot):
            off, sz = schedule[i]
            return pltpu.make_async_copy(
                x_hbm.at[pl.ds(off, sz), :], xbuf.at[slot, pl.ds(0, sz), :],
                load_sem.at[slot])

        def store(i, slot):
            off, sz = schedule[i]
            return pltpu.make_async_copy(
                obuf.at[slot, pl.ds(0, sz), :], o_hbm.at[pl.ds(off, sz), :],
                store_sem.at[slot])

        def w_half(h):
            return pltpu.make_async_copy(
                w_hbm.at[pl.ds(h * Eh, Eh), :], wv.at[pl.ds(h * Eh, Eh), :],
                w_sem.at[h])

        w_half(0).start()
        load(0, 0).start()
        pltpu.make_async_copy(b_hbm, bv, b_sem).start()
        w_half(1).start()
        for i in range(1, min(_NBUF, n)):
            load(i, i % _NBUF).start()
        pltpu.make_async_copy(b_hbm, bv, b_sem).wait()

        sz0 = schedule[0][1]
        load(0, 0).wait()
        w_half(0).wait()
        acc = jax.lax.dot_general(
            xbuf[0, 0:sz0, :], wv[0:Eh, :], dims,
            preferred_element_type=jnp.float32)
        obuf[0, 0:sz0, 0:Eh] = acc + bv[:, 0:Eh]
        w_half(1).wait()
        acc = jax.lax.dot_general(
            xbuf[0, 0:sz0, :], wv[Eh:E, :], dims,
            preferred_element_type=jnp.float32)
        obuf[0, 0:sz0, Eh:E] = acc + bv[:, Eh:E]
        store(0, 0).start()
        if _NBUF < n:
            load(_NBUF, 0).start()

        for i in range(1, n):
            slot = i % _NBUF
            sz = schedule[i][1]
            load(i, slot).wait()
            if i >= _NBUF:
                store(i - _NBUF, slot).wait()
            acc = jax.lax.dot_general(
                xbuf[slot, 0:sz, :], wv[...], dims,
                preferred_element_type=jnp.float32)
            obuf[slot, 0:sz, :] = acc + bv[...]
            store(i, slot).start()
            nxt = i + _NBUF
            if nxt < n:
                load(nxt, slot).start()

        for i in range(max(0, n - _NBUF), n):
            store(i, i % _NBUF).wait()

    return body


def kernel(src, W, b):
    B, S, D = src.shape
    E = W.shape[0]
    M = B * S
    x = src.reshape(M, D)
    schedule = _chunk_schedule(M)
    bufrows = max(sz for _, sz in schedule)

    out = pl.pallas_call(
        _make_body(schedule, E),
        in_specs=[
            pl.BlockSpec(memory_space=pltpu.HBM),
            pl.BlockSpec(memory_space=pltpu.HBM),
            pl.BlockSpec(memory_space=pltpu.HBM),
        ],
        out_specs=pl.BlockSpec(memory_space=pltpu.HBM),
        out_shape=jax.ShapeDtypeStruct((M, E), jnp.float32),
        scratch_shapes=[
            pltpu.VMEM((_NBUF, bufrows, D), jnp.float32),
            pltpu.VMEM((_NBUF, bufrows, E), jnp.float32),
            pltpu.VMEM((E, D), jnp.float32),
            pltpu.VMEM((1, E), jnp.float32),
            pltpu.SemaphoreType.DMA((_NBUF,)),
            pltpu.SemaphoreType.DMA((_NBUF,)),
            pltpu.SemaphoreType.DMA((2,)),
            pltpu.SemaphoreType.DMA,
        ],
    )(x, W, b.reshape(1, E))
    return out.reshape(B, S, E)


# speedup vs baseline: 1.0238x; 1.0063x over previous
import jax
import jax.numpy as jnp
from jax.experimental import pallas as pl
from jax.experimental.pallas import tpu as pltpu

_CH = 2048
_NBUF = 4


def _chunk_schedule(M):
    head = [256, 256, 512, 1024]
    tail = [1024, 512, 256, 256]
    mid_total = M - sum(head) - sum(tail)
    if mid_total > 0 and mid_total % _CH == 0:
        sizes = head + [_CH] * (mid_total // _CH) + tail
    elif M % _CH == 0:
        sizes = [_CH] * (M // _CH)
    else:
        sizes = [M]
    offs, o = [], 0
    for s in sizes:
        offs.append(o)
        o += s
    return list(zip(offs, sizes))


def _make_body(schedule):
    n = len(schedule)

    def body(x_hbm, w_hbm, b_hbm, o_hbm,
             xbuf, obuf, wv, bv, load_sem, store_sem, w_sem, b_sem):
        def load(i, slot):
            off, sz = schedule[i]
            return pltpu.make_async_copy(
                x_hbm.at[pl.ds(off, sz), :], xbuf.at[slot, pl.ds(0, sz), :],
                load_sem.at[slot])

        def store(i, slot):
            off, sz = schedule[i]
            return pltpu.make_async_copy(
                obuf.at[slot, pl.ds(0, sz), :], o_hbm.at[pl.ds(off, sz), :],
                store_sem.at[slot])

        pltpu.make_async_copy(w_hbm, wv, w_sem).start()
        pltpu.make_async_copy(b_hbm, bv, b_sem).start()
        for i in range(min(_NBUF, n)):
            load(i, i % _NBUF).start()
        pltpu.make_async_copy(w_hbm, wv, w_sem).wait()
        pltpu.make_async_copy(b_hbm, bv, b_sem).wait()

        for i in range(n):
            slot = i % _NBUF
            sz = schedule[i][1]
            load(i, slot).wait()
            if i >= _NBUF:
                store(i - _NBUF, slot).wait()
            acc = jax.lax.dot_general(
                xbuf[slot, 0:sz, :], wv[...],
                dimension_numbers=(((1,), (1,)), ((), ())),
                preferred_element_type=jnp.float32,
            )
            obuf[slot, 0:sz, :] = acc + bv[...]
            store(i, slot).start()
            nxt = i + _NBUF
            if nxt < n:
                load(nxt, slot).start()

        for i in range(max(0, n - _NBUF), n):
            store(i, i % _NBUF).wait()

    return body


def kernel(src, W, b):
    B, S, D = src.shape
    E = W.shape[0]
    M = B * S
    x = src.reshape(M, D)
    schedule = _chunk_schedule(M)
    bufrows = max(sz for _, sz in schedule)

    out = pl.pallas_call(
        _make_body(schedule),
        in_specs=[
            pl.BlockSpec(memory_space=pltpu.HBM),
            pl.BlockSpec(memory_space=pltpu.HBM),
            pl.BlockSpec(memory_space=pltpu.HBM),
        ],
        out_specs=pl.BlockSpec(memory_space=pltpu.HBM),
        out_shape=jax.ShapeDtypeStruct((M, E), jnp.float32),
        scratch_shapes=[
            pltpu.VMEM((_NBUF, bufrows, D), jnp.float32),
            pltpu.VMEM((_NBUF, bufrows, E), jnp.float32),
            pltpu.VMEM((E, D), jnp.float32),
            pltpu.VMEM((1, E), jnp.float32),
            pltpu.SemaphoreType.DMA((_NBUF,)),
            pltpu.SemaphoreType.DMA((_NBUF,)),
            pltpu.SemaphoreType.DMA,
            pltpu.SemaphoreType.DMA,
        ],
    )(x, W, b.reshape(1, E))
    return out.reshape(B, S, E)
